# baseline (speedup 1.0000x reference)
import jax
import jax.numpy as jnp
from jax.experimental import pallas as pl
from jax.experimental.pallas import tpu as pltpu

E = 8
C = 1024
M = 1024
FF = 4096
S = 4096
FFB = 2048
SB = 512


def _ffn_body(x_ref, w1_ref, w2_ref, out_ref, acc_ref):
    j = pl.program_id(1)
    h = jnp.dot(x_ref[0].astype(jnp.bfloat16), w1_ref[0].astype(jnp.bfloat16),
                preferred_element_type=jnp.float32)
    h = jax.nn.gelu(h)
    part = jnp.dot(h.astype(jnp.bfloat16), w2_ref[0].astype(jnp.bfloat16),
                   preferred_element_type=jnp.float32)

    @pl.when(j == 0)
    def _():
        acc_ref[...] = part

    @pl.when(j != 0)
    def _():
        acc_ref[...] += part

    @pl.when(j == FF // FFB - 1)
    def _():
        out_ref[0] = acc_ref[...].astype(jnp.bfloat16)


def _ffn(x, w1, w2):
    return pl.pallas_call(
        _ffn_body,
        grid=(E, FF // FFB),
        in_specs=[
            pl.BlockSpec((1, C, M), lambda e, j: (e, 0, 0)),
            pl.BlockSpec((1, M, FFB), lambda e, j: (e, 0, j)),
            pl.BlockSpec((1, FFB, M), lambda e, j: (e, j, 0)),
        ],
        out_specs=pl.BlockSpec((1, C, M), lambda e, j: (e, 0, 0)),
        out_shape=jax.ShapeDtypeStruct((E, C, M), jnp.bfloat16),
        scratch_shapes=[pltpu.VMEM((C, M), jnp.float32)],
        compiler_params=pltpu.CompilerParams(
            dimension_semantics=("parallel", "arbitrary"),
            vmem_limit_bytes=100 * 1024 * 1024,
        ),
    )(x, w1, w2)


def _combine_body(cw_ref, eo_ref, out_ref):
    out_ref[...] = jnp.dot(cw_ref[...], eo_ref[...],
                           preferred_element_type=jnp.float32)


def _combine(cw2, eo2):
    return pl.pallas_call(
        _combine_body,
        grid=(S // SB,),
        in_specs=[
            pl.BlockSpec((SB, E * C), lambda i: (i, 0)),
            pl.BlockSpec((E * C, M), lambda i: (0, 0)),
        ],
        out_specs=pl.BlockSpec((SB, M), lambda i: (i, 0)),
        out_shape=jax.ShapeDtypeStruct((S, M), jnp.float32),
        compiler_params=pltpu.CompilerParams(
            vmem_limit_bytes=100 * 1024 * 1024,
        ),
    )(cw2, eo2)


def kernel(inputs, w1, w2, combine_weights):
    x = inputs[: E * C].reshape(E, C, M)
    eo = _ffn(x, w1, w2)
    cwb = combine_weights.reshape(S, E * C).astype(jnp.bfloat16)
    out = _combine(cwb, eo.reshape(E * C, M))
    return out.reshape(2, 2048, M)


# speedup vs baseline: 1.0844x; 1.0844x over previous
import jax
import jax.numpy as jnp
from jax.experimental import pallas as pl
from jax.experimental.pallas import tpu as pltpu

E = 8
C = 1024
M = 1024
FF = 4096
S = 4096
FFB = 2048
SB = 512


def _ffn_body(x_ref, w1_ref, w2_ref, out_ref, acc_ref):
    j = pl.program_id(1)
    h = jnp.dot(x_ref[0].astype(jnp.bfloat16), w1_ref[0].astype(jnp.bfloat16),
                preferred_element_type=jnp.float32)
    h = jax.nn.gelu(h)
    part = jnp.dot(h.astype(jnp.bfloat16), w2_ref[0].astype(jnp.bfloat16),
                   preferred_element_type=jnp.float32)

    @pl.when(j == 0)
    def _():
        acc_ref[...] = part

    @pl.when(j != 0)
    def _():
        acc_ref[...] += part

    @pl.when(j == FF // FFB - 1)
    def _():
        out_ref[0] = acc_ref[...].astype(jnp.bfloat16)


def _ffn(x, w1, w2):
    return pl.pallas_call(
        _ffn_body,
        grid=(E, FF // FFB),
        in_specs=[
            pl.BlockSpec((1, C, M), lambda e, j: (e, 0, 0)),
            pl.BlockSpec((1, M, FFB), lambda e, j: (e, 0, j)),
            pl.BlockSpec((1, FFB, M), lambda e, j: (e, j, 0)),
        ],
        out_specs=pl.BlockSpec((1, C, M), lambda e, j: (e, 0, 0)),
        out_shape=jax.ShapeDtypeStruct((E, C, M), jnp.bfloat16),
        scratch_shapes=[pltpu.VMEM((C, M), jnp.float32)],
        compiler_params=pltpu.CompilerParams(
            dimension_semantics=("parallel", "arbitrary"),
            vmem_limit_bytes=100 * 1024 * 1024,
        ),
    )(x, w1, w2)


def _combine_body(cw_ref, eo_ref, out_ref):
    cwb = cw_ref[...].astype(jnp.bfloat16)
    out_ref[...] = jnp.dot(cwb, eo_ref[...], preferred_element_type=jnp.float32)


def _combine(cw2, eo2):
    return pl.pallas_call(
        _combine_body,
        grid=(S // SB,),
        in_specs=[
            pl.BlockSpec((SB, E * C), lambda i: (i, 0)),
            pl.BlockSpec((E * C, M), lambda i: (0, 0)),
        ],
        out_specs=pl.BlockSpec((SB, M), lambda i: (i, 0)),
        out_shape=jax.ShapeDtypeStruct((S, M), jnp.float32),
        compiler_params=pltpu.CompilerParams(
            vmem_limit_bytes=100 * 1024 * 1024,
        ),
    )(cw2, eo2)


def kernel(inputs, w1, w2, combine_weights):
    x = inputs[: E * C].reshape(E, C, M)
    eo = _ffn(x, w1, w2)
    out = _combine(combine_weights.reshape(S, E * C), eo.reshape(E * C, M))
    return out.reshape(2, 2048, M)
